# Initial kernel scaffold; baseline (speedup 1.0000x reference)
import jax
import jax.numpy as jnp
from jax.experimental import pallas as pl


def kernel(x, edge_index, edge_attr, W1_rel, b1, W1_root, W2_rel, b2, W2_root, Wfc1, bfc1, Wfc2, bfc2):
    raise NotImplementedError("write your pallas kernel here")


# speedup vs baseline: 1.0424x; 1.0424x over previous
import functools

import jax
import jax.numpy as jnp
from jax import lax
from jax.experimental import pallas as pl
from jax.experimental.pallas import tpu as pltpu
from jax.experimental.pallas import tpu_sc as plsc

N = 10000
E = 160000
NC = 2
NS = 16
L = 16
NW = NC * NS
NPW = 320
NPAD = NW * NPW
ACCR = 324
PADROW = 321
EB = 3200
NG = EB // L
NCH = E // EB


def _segmax_body(D, table, ei, w, out, dstb, srcb, wb, msrc, mw, mdstl,
                 rows, acc, sem):
    wid = lax.axis_index("s") * NC + lax.axis_index("c")
    lo = wid * NPW
    iota = lax.iota(jnp.int32, L)
    lov = lax.broadcast(lo, (L,))
    hiv = lov + NPW
    ninf = jnp.full((L,), -jnp.inf, dtype=jnp.float32)
    nf = D // L

    def init_body(i, _):
        acc[pl.ds(i * L, L)] = ninf
        return 0
    lax.fori_loop(0, ACCR * nf, init_body, 0)

    def chunk_body(c, _):
        base = c * EB
        pltpu.sync_copy(ei.at[1, pl.ds(base, EB)], dstb)
        pltpu.sync_copy(ei.at[0, pl.ds(base, EB)], srcb)
        pltpu.sync_copy(w.at[pl.ds(base, EB)], wb)

        def fbody(g, off):
            s = g * L
            dv = dstb[pl.ds(s, L)]
            sv = srcb[pl.ds(s, L)]
            wv = wb[pl.ds(s, L)]
            msk = (dv >= lov) & (dv < hiv)
            mi = msk.astype(jnp.int32)
            pos = jnp.cumsum(mi) - mi + off
            plsc.store_scatter(msrc, [pos], sv, mask=msk)
            plsc.store_scatter(mw, [pos], wv, mask=msk)
            plsc.store_scatter(mdstl, [pos], (dv - lov) * D, mask=msk)
            return off + jnp.sum(mi)

        m = lax.fori_loop(0, NG, fbody, jnp.int32(0))
        pidx = iota + m
        plsc.store_scatter(msrc, [pidx], jnp.zeros((L,), jnp.int32))
        plsc.store_scatter(mw, [pidx], jnp.zeros((L,), jnp.float32))
        plsc.store_scatter(mdstl, [pidx],
                           jnp.full((L,), PADROW * D, jnp.int32))

        ng = (m + L - 1) // L

        def gbody(g, _):
            gb = g * L
            idxv = msrc[pl.ds(gb, L)]
            pltpu.async_copy(table.at[idxv], rows, sem).wait()
            for j in range(L):
                ev = iota * 0 + (gb + j)
                wj = plsc.load_gather(mw, [ev])
                ab = plsc.load_gather(mdstl, [ev]) + iota
                for f in range(nf):
                    addr = ab + (f * L)
                    cur = plsc.load_gather(acc, [addr])
                    val = rows[j, pl.ds(f * L, L)] * wj
                    plsc.store_scatter(acc, [addr], jnp.maximum(cur, val))
            return 0

        lax.fori_loop(0, ng, gbody, 0)
        return 0

    lax.fori_loop(0, NCH, chunk_body, 0)
    pltpu.sync_copy(acc.at[pl.ds(0, NPW * D)], out.at[pl.ds(lo * D, NPW * D)])


def _make_segmax(D):
    mesh = plsc.VectorSubcoreMesh(core_axis_name="c", subcore_axis_name="s",
                                  num_cores=NC, num_subcores=NS)
    body = functools.partial(_segmax_body, D)

    @functools.partial(
        pl.kernel,
        mesh=mesh,
        out_type=jax.ShapeDtypeStruct((NPAD * D,), jnp.float32),
        scratch_types=[
            pltpu.VMEM((EB,), jnp.int32),
            pltpu.VMEM((EB,), jnp.int32),
            pltpu.VMEM((EB,), jnp.float32),
            pltpu.VMEM((EB + L,), jnp.int32),
            pltpu.VMEM((EB + L,), jnp.float32),
            pltpu.VMEM((EB + L,), jnp.int32),
            pltpu.VMEM((L, D), jnp.float32),
            pltpu.VMEM((ACCR * D,), jnp.float32),
            pltpu.SemaphoreType.DMA,
        ],
        compiler_params=pltpu.CompilerParams(needs_layout_passes=False,
                                             use_tc_tiling_on_sc=False),
    )
    def k(table, ei, w, out, dstb, srcb, wb, msrc, mw, mdstl, rows, acc, sem):
        body(table, ei, w, out, dstb, srcb, wb, msrc, mw, mdstl, rows, acc,
             sem)

    return k


_segmax256 = _make_segmax(256)
_segmax32 = _make_segmax(32)


def _elu(h):
    return jnp.where(h > 0, h, jnp.exp(h) - 1.0)


def _fixagg(a):
    return jnp.where(a == -jnp.inf, 0.0, a)


def _dense1_kernel(agg_ref, x_ref, wr_ref, b_ref, wroot_ref, o_ref):
    agg = _fixagg(agg_ref[...])
    h = (jnp.dot(agg, wr_ref[...], preferred_element_type=jnp.float32)
         + b_ref[...]
         + jnp.dot(x_ref[...], wroot_ref[...],
                   preferred_element_type=jnp.float32))
    o_ref[...] = _elu(h)


def _dense2_kernel(agg_ref, h1_ref, wr_ref, b_ref, wroot_ref,
                   wfc1_ref, bfc1_ref, wfc2_ref, bfc2_ref, o_ref):
    agg = _fixagg(agg_ref[...])
    h2 = _elu(jnp.dot(agg, wr_ref[...], preferred_element_type=jnp.float32)
              + b_ref[...]
              + jnp.dot(h1_ref[...], wroot_ref[...],
                        preferred_element_type=jnp.float32))
    h3 = _elu(jnp.dot(h2, wfc1_ref[...], preferred_element_type=jnp.float32)
              + bfc1_ref[...])
    o_ref[...] = (jnp.dot(h3, wfc2_ref[...], preferred_element_type=jnp.float32)
                  + bfc2_ref[...])


_RB = 1024
_GRID = (NPAD + _RB - 1) // _RB


def _dense1(agg, x, W1_rel, b1, W1_root):
    return pl.pallas_call(
        _dense1_kernel,
        grid=(_GRID,),
        in_specs=[
            pl.BlockSpec((_RB, 256), lambda i: (i, 0)),
            pl.BlockSpec((_RB, 256), lambda i: (i, 0)),
            pl.BlockSpec((256, 32), lambda i: (0, 0)),
            pl.BlockSpec((1, 32), lambda i: (0, 0)),
            pl.BlockSpec((256, 32), lambda i: (0, 0)),
        ],
        out_specs=pl.BlockSpec((_RB, 32), lambda i: (i, 0)),
        out_shape=jax.ShapeDtypeStruct((NPAD, 32), jnp.float32),
    )(agg, x, W1_rel, b1, W1_root)


def _dense2(agg2, h1, W2_rel, b2, W2_root, Wfc1, bfc1, Wfc2, bfc2):
    return pl.pallas_call(
        _dense2_kernel,
        grid=(_GRID,),
        in_specs=[
            pl.BlockSpec((_RB, 32), lambda i: (i, 0)),
            pl.BlockSpec((_RB, 32), lambda i: (i, 0)),
            pl.BlockSpec((32, 64), lambda i: (0, 0)),
            pl.BlockSpec((1, 64), lambda i: (0, 0)),
            pl.BlockSpec((32, 64), lambda i: (0, 0)),
            pl.BlockSpec((64, 128), lambda i: (0, 0)),
            pl.BlockSpec((1, 128), lambda i: (0, 0)),
            pl.BlockSpec((128, 16), lambda i: (0, 0)),
            pl.BlockSpec((1, 16), lambda i: (0, 0)),
        ],
        out_specs=pl.BlockSpec((_RB, 16), lambda i: (i, 0)),
        out_shape=jax.ShapeDtypeStruct((NPAD, 16), jnp.float32),
    )(agg2, h1, W2_rel, b2, W2_root, Wfc1, bfc1, Wfc2, bfc2)


def kernel(x, edge_index, edge_attr, W1_rel, b1, W1_root, W2_rel, b2, W2_root,
           Wfc1, bfc1, Wfc2, bfc2):
    agg1 = _segmax256(x, edge_index, edge_attr).reshape(NPAD, 256)
    h1 = _dense1(agg1, x, W1_rel, b1.reshape(1, 32), W1_root)
    agg2 = _segmax32(h1, edge_index, edge_attr).reshape(NPAD, 32)
    out = _dense2(agg2, h1, W2_rel, b2.reshape(1, 64), W2_root,
                  Wfc1, bfc1.reshape(1, 128), Wfc2, bfc2.reshape(1, 16))
    return out[:N]
